# baseline (speedup 1.0000x reference)
import jax
import jax.numpy as jnp
from jax.experimental import pallas as pl
from jax.experimental.pallas import tpu as pltpu

N_ROWS = 4096
N_COLS = 200
LANES = 128
IC = 40


def _f32_sort_key(x):
    b = jax.lax.bitcast_convert_type(x, jnp.int32)
    return b ^ ((b >> 31) & jnp.int32(0x7FFFFFFF))


def _listmle_block_kernel(t_ref, p_ref, out_ref):
    tT = t_ref[:].T
    pT = p_ref[:].T
    kT = _f32_sort_key(tT)

    m = jnp.max(pT, axis=0, keepdims=True)
    shT = pT - m
    eT = jnp.exp(shT)
    e3 = eT[None, :, :]
    k3 = kT[None, :, :]

    acc = jnp.zeros((1, LANES), jnp.float32) - jnp.sum(shT, axis=0,
                                                       keepdims=True)
    for ci in range(N_COLS // IC):
        i0 = ci * IC
        i1 = i0 + IC
        ki3 = kT[i0:i1][:, None, :]
        ki3t = ki3 + 1
        s = jnp.zeros((IC, LANES), jnp.float32)
        if i0 > 0:
            s += jnp.sum(jnp.where(k3[:, :i0] < ki3, e3[:, :i0], 0.0),
                         axis=1)
        if i1 < N_COLS:
            s += jnp.sum(jnp.where(k3[:, i1:] < ki3t, e3[:, i1:], 0.0),
                         axis=1)
        ii = jax.lax.broadcasted_iota(jnp.int32, (IC, IC, 1), 0)
        jj = jax.lax.broadcasted_iota(jnp.int32, (IC, IC, 1), 1)
        kiL = jnp.where(jj >= ii, ki3t, ki3)
        s += jnp.sum(jnp.where(k3[:, i0:i1] < kiL, e3[:, i0:i1], 0.0),
                     axis=1)
        acc += jnp.sum(jnp.log(s), axis=0, keepdims=True)

    out_ref[:, :] = jnp.concatenate(
        [acc, jnp.zeros((7, LANES), jnp.float32)], axis=0)


@jax.jit
def kernel(preds, targets):
    grid = (N_ROWS // LANES,)
    acc = pl.pallas_call(
        _listmle_block_kernel,
        grid=grid,
        in_specs=[
            pl.BlockSpec((LANES, N_COLS), lambda b: (b, 0)),
            pl.BlockSpec((LANES, N_COLS), lambda b: (b, 0)),
        ],
        out_specs=pl.BlockSpec((8, LANES), lambda b: (b, 0)),
        out_shape=jax.ShapeDtypeStruct((N_ROWS // LANES * 8, LANES),
                                       jnp.float32),
        compiler_params=pltpu.CompilerParams(
            dimension_semantics=("parallel",)),
    )(targets, preds)
    return jnp.sum(acc) / N_ROWS


# speedup vs baseline: 1.1313x; 1.1313x over previous
import jax
import jax.numpy as jnp
from jax.experimental import pallas as pl

N_ROWS = 4096
N_COLS = 200
LANES = 128
IC = 40


def _f32_sort_key(x):
    b = jax.lax.bitcast_convert_type(x, jnp.int32)
    return b ^ ((b >> 31) & jnp.int32(0x7FFFFFFF))


def _listmle_block_kernel(tT_ref, pT_ref, out_ref):
    tT = tT_ref[:]
    pT = pT_ref[:]
    kT = _f32_sort_key(tT)

    m = jnp.max(pT, axis=0, keepdims=True)
    shT = pT - m
    eT = jnp.exp(shT)
    e3 = eT[None, :, :]
    k3 = kT[None, :, :]

    acc = jnp.zeros((1, LANES), jnp.float32) - jnp.sum(shT, axis=0,
                                                       keepdims=True)
    for ci in range(N_COLS // IC):
        i0 = ci * IC
        i1 = i0 + IC
        ki3 = kT[i0:i1][:, None, :]
        ki3t = ki3 + 1
        s = jnp.zeros((IC, LANES), jnp.float32)
        if i0 > 0:
            s += jnp.sum(jnp.where(k3[:, :i0] < ki3, e3[:, :i0], 0.0),
                         axis=1)
        if i1 < N_COLS:
            s += jnp.sum(jnp.where(k3[:, i1:] < ki3t, e3[:, i1:], 0.0),
                         axis=1)
        ii = jax.lax.broadcasted_iota(jnp.int32, (IC, IC, 1), 0)
        jj = jax.lax.broadcasted_iota(jnp.int32, (IC, IC, 1), 1)
        kiL = jnp.where(jj >= ii, ki3t, ki3)
        s += jnp.sum(jnp.where(k3[:, i0:i1] < kiL, e3[:, i0:i1], 0.0),
                     axis=1)
        acc += jnp.sum(jnp.log(s), axis=0, keepdims=True)

    @pl.when(pl.program_id(0) == 0)
    def _init():
        out_ref[:, :] = jnp.zeros((1, LANES), jnp.float32)

    out_ref[:, :] += acc


@jax.jit
def kernel(preds, targets):
    pT = preds.T
    tT = targets.T
    grid = (N_ROWS // LANES,)
    acc = pl.pallas_call(
        _listmle_block_kernel,
        grid=grid,
        in_specs=[
            pl.BlockSpec((N_COLS, LANES), lambda b: (0, b)),
            pl.BlockSpec((N_COLS, LANES), lambda b: (0, b)),
        ],
        out_specs=pl.BlockSpec((1, LANES), lambda b: (0, 0)),
        out_shape=jax.ShapeDtypeStruct((1, LANES), jnp.float32),
    )(tT, pT)
    return jnp.sum(acc) / N_ROWS
